# baseline (speedup 1.0000x reference)
import functools
import jax
import jax.numpy as jnp
from jax import lax
from jax.experimental import pallas as pl
from jax.experimental.pallas import tpu as pltpu
from jax.experimental.pallas import tpu_sc as plsc

N_NODES = 10000
N_EDGES = 320000
D = 128
G = 128

NC = 2
NS = 16
NW = NC * NS

K = 128
NCH = 80
E_PER_W = NCH * K
E_PAD = NW * E_PER_W
NPR = 10240

NSEG = 4
SEGW = 2560
RPW = 78 * K
SEG_CH = (20, 20, 20, 18)
SEG_LEN = tuple(c * K for c in SEG_CH)
XTRA = N_EDGES - NW * RPW

BLK = 1024
N_BLOCKS = NPR // BLK


_sc_mesh = plsc.VectorSubcoreMesh(core_axis_name="c", subcore_axis_name="s")


@functools.partial(
    pl.kernel,
    out_type=jax.ShapeDtypeStruct((NC, NPR, D), jnp.float32),
    mesh=_sc_mesh,
    scratch_types=[
        pltpu.VMEM((2, 2, SEGW), jnp.int32),
        pltpu.VMEM((2, K, D), jnp.float32),
        pltpu.VMEM_SHARED((NPR, D), jnp.float32),
        pltpu.SemaphoreType.DMA,
        pltpu.SemaphoreType.DMA,
        pltpu.SemaphoreType.DMA,
        pltpu.SemaphoreType.DMA,
    ],
)
def _sc_scatter_add(x_hbm, ei_hbm, out_hbm,
                    seg_v, rows_v, agg_sh, gsem0, gsem1, isem0, isem1):
    cid = lax.axis_index("c")
    sid = lax.axis_index("s")
    wid = cid * NS + sid
    gsems = (gsem0, gsem1)
    isems = (isem0, isem1)
    base = wid * RPW

    def seg_dma(u, slot, sem):
        n = SEG_LEN[u]
        return pltpu.make_async_copy(
            ei_hbm.at[:, pl.ds(base + u * SEGW, n)],
            seg_v.at[slot, :, pl.ds(0, n)], sem)

    seg_dma(0, 0, isem0).start()

    @pl.loop(0, K)
    def _(r):
        for c in range(D // 16):
            rows_v[0, r, pl.ds(c * 16, 16)] = jnp.zeros((16,), jnp.float32)

    zrows = NPR // NS
    half = NS // 2
    xlo = cid * half
    is_x_tile = (sid >= xlo) & (sid < xlo + half)
    tail_tile = (cid == 1) & (sid == NS - 1)

    @pl.when(is_x_tile & ~tail_tile)
    def _():
        pltpu.sync_copy(x_hbm.at[pl.ds(sid * zrows, zrows)],
                        agg_sh.at[pl.ds(sid * zrows, zrows)])

    @pl.when(tail_tile)
    def _():
        pltpu.sync_copy(x_hbm.at[pl.ds(9600, 400)],
                        agg_sh.at[pl.ds(9600, 400)])
        pltpu.sync_copy(rows_v.at[0], agg_sh.at[pl.ds(10000, K)])
        pltpu.sync_copy(rows_v.at[0, pl.ds(0, 112)],
                        agg_sh.at[pl.ds(10128, 112)])

    @pl.when(~is_x_tile)
    def _():
        for q in range(zrows // K):
            pltpu.sync_copy(rows_v.at[0],
                            agg_sh.at[pl.ds(sid * zrows + q * K, K)])

    plsc.subcore_barrier()

    def gather(slot, t, rb, sem):
        return pltpu.async_copy(
            x_hbm.at[seg_v.at[slot, 0, pl.ds(t * K, K)]],
            rows_v.at[rb], sem)

    def scatter(slot, t, rb):
        pltpu.sync_copy(rows_v.at[rb],
                        agg_sh.at[seg_v.at[slot, 1, pl.ds(t * K, K)]],
                        add=True)

    seg_dma(0, 0, isem0).wait()
    seg_dma(1, 1, isem1).start()

    for u in range(NSEG):
        b = u % 2
        if u > 0:
            seg_dma(u, b, isems[b]).wait()
            if u + 1 < NSEG:
                seg_dma(u + 1, 1 - b, isems[1 - b]).start()
        gather(b, 0, 0, gsem0)
        gather(b, 1, 1, gsem1)

        @pl.loop(0, SEG_CH[u] // 2 - 1)
        def _(it):
            for rb in range(2):
                t = 2 * it + rb
                pltpu.make_async_copy(
                    x_hbm.at[seg_v.at[b, 0, pl.ds(t * K, K)]],
                    rows_v.at[rb], gsems[rb]).wait()
                scatter(b, t, rb)
                gather(b, t + 2, rb, gsems[rb])

        for rb in range(2):
            t = SEG_CH[u] - 2 + rb
            pltpu.make_async_copy(
                x_hbm.at[seg_v.at[b, 0, pl.ds(t * K, K)]],
                rows_v.at[rb], gsems[rb]).wait()
            scatter(b, t, rb)

    q = sid * NC + cid

    @pl.when(sid < (XTRA // K) // NC)
    def _():
        pltpu.make_async_copy(
            ei_hbm.at[:, pl.ds(NW * RPW + q * K, K)],
            seg_v.at[0, :, pl.ds(0, K)], isem0).start()
        pltpu.make_async_copy(
            ei_hbm.at[:, pl.ds(NW * RPW + q * K, K)],
            seg_v.at[0, :, pl.ds(0, K)], isem0).wait()
        pltpu.make_async_copy(
            x_hbm.at[seg_v.at[0, 0, pl.ds(0, K)]],
            rows_v.at[0], gsem0).start()
        pltpu.make_async_copy(
            x_hbm.at[seg_v.at[0, 0, pl.ds(0, K)]],
            rows_v.at[0], gsem0).wait()
        scatter(0, 0, 0)

    plsc.subcore_barrier()

    pltpu.sync_copy(agg_sh.at[pl.ds(sid * zrows, zrows)],
                    out_hbm.at[cid, pl.ds(sid * zrows, zrows)])


def _tc_body(agg_ref, batch_ref, W1_ref, b1_ref, W2_ref, b2_ref,
             Wlin_ref, blin_ref, out_ref, pool_acc):
    i = pl.program_id(0)
    h = agg_ref[0] + agg_ref[1]
    z = jnp.dot(h, W1_ref[...], preferred_element_type=jnp.float32)
    z = jnp.maximum(z + b1_ref[...], 0.0)
    h2 = jnp.dot(z, W2_ref[...],
                 preferred_element_type=jnp.float32) + b2_ref[...]
    b = batch_ref[0]
    gids = lax.broadcasted_iota(jnp.int32, (G, BLK), 0)
    mask = (gids == b).astype(jnp.float32)
    seg = jnp.dot(mask, h2, preferred_element_type=jnp.float32,
                  precision=lax.Precision.HIGHEST)

    @pl.when(i == 0)
    def _():
        pool_acc[...] = jnp.zeros_like(pool_acc)
        out_ref[...] = jnp.zeros_like(out_ref)

    pool_acc[...] += seg

    @pl.when(i == pl.num_programs(0) - 1)
    def _():
        out_ref[...] = jnp.dot(
            pool_acc[...], Wlin_ref[...],
            preferred_element_type=jnp.float32) + blin_ref[...]


_tc_call = pl.pallas_call(
    _tc_body,
    grid=(N_BLOCKS,),
    in_specs=[
        pl.BlockSpec((NC, BLK, D), lambda i: (0, i, 0)),
        pl.BlockSpec((1, 1, BLK), lambda i: (i, 0, 0)),
        pl.BlockSpec((D, D), lambda i: (0, 0)),
        pl.BlockSpec((1, D), lambda i: (0, 0)),
        pl.BlockSpec((D, D), lambda i: (0, 0)),
        pl.BlockSpec((1, D), lambda i: (0, 0)),
        pl.BlockSpec((D, 1), lambda i: (0, 0)),
        pl.BlockSpec((1, 1), lambda i: (0, 0)),
    ],
    out_specs=pl.BlockSpec((G, 1), lambda i: (0, 0)),
    out_shape=jax.ShapeDtypeStruct((G, 1), jnp.float32),
    scratch_shapes=[pltpu.VMEM((G, D), jnp.float32)],
)


def kernel(x, edge_index, batch, W1, b1, W2, b2, Wlin, blin):
    ei = edge_index.astype(jnp.int32)
    agg = _sc_scatter_add(x, ei)

    batch_p = jnp.pad(batch.astype(jnp.int32), (0, NPR - N_NODES),
                      constant_values=G)
    batch3 = batch_p.reshape(N_BLOCKS, 1, BLK)
    out = _tc_call(agg, batch3, W1, b1.reshape(1, D), W2,
                   b2.reshape(1, D), Wlin, blin.reshape(1, 1))
    return out


# speedup vs baseline: 1.0120x; 1.0120x over previous
import functools
import jax
import jax.numpy as jnp
from jax import lax
from jax.experimental import pallas as pl
from jax.experimental.pallas import tpu as pltpu
from jax.experimental.pallas import tpu_sc as plsc

N_NODES = 10000
N_EDGES = 320000
D = 128
G = 128

NC = 2
NS = 16
NW = NC * NS

K = 128
NCH = 80
E_PER_W = NCH * K
E_PAD = NW * E_PER_W
NPR = 10240

NSEG = 4
SEGW = 2560
RPW = 78 * K
SEG_CH = (20, 20, 20, 18)
SEG_LEN = tuple(c * K for c in SEG_CH)
XTRA = N_EDGES - NW * RPW

BLK = 2560
N_BLOCKS = NPR // BLK


_sc_mesh = plsc.VectorSubcoreMesh(core_axis_name="c", subcore_axis_name="s")


@functools.partial(
    pl.kernel,
    out_type=jax.ShapeDtypeStruct((NC, NPR, D), jnp.float32),
    mesh=_sc_mesh,
    scratch_types=[
        pltpu.VMEM((2, 2, SEGW), jnp.int32),
        pltpu.VMEM((2, K, D), jnp.float32),
        pltpu.VMEM_SHARED((NPR, D), jnp.float32),
        pltpu.SemaphoreType.DMA,
        pltpu.SemaphoreType.DMA,
        pltpu.SemaphoreType.DMA,
        pltpu.SemaphoreType.DMA,
    ],
)
def _sc_scatter_add(x_hbm, ei_hbm, out_hbm,
                    seg_v, rows_v, agg_sh, gsem0, gsem1, isem0, isem1):
    cid = lax.axis_index("c")
    sid = lax.axis_index("s")
    wid = cid * NS + sid
    gsems = (gsem0, gsem1)
    isems = (isem0, isem1)
    base = wid * RPW

    def seg_dma(u, slot, sem):
        n = SEG_LEN[u]
        return pltpu.make_async_copy(
            ei_hbm.at[:, pl.ds(base + u * SEGW, n)],
            seg_v.at[slot, :, pl.ds(0, n)], sem)

    seg_dma(0, 0, isem0).start()

    @pl.loop(0, K)
    def _(r):
        for c in range(D // 16):
            rows_v[0, r, pl.ds(c * 16, 16)] = jnp.zeros((16,), jnp.float32)

    zrows = NPR // NS
    half = NS // 2
    xlo = cid * half
    is_x_tile = (sid >= xlo) & (sid < xlo + half)
    tail_tile = (cid == 1) & (sid == NS - 1)

    @pl.when(is_x_tile & ~tail_tile)
    def _():
        pltpu.sync_copy(x_hbm.at[pl.ds(sid * zrows, zrows)],
                        agg_sh.at[pl.ds(sid * zrows, zrows)])

    @pl.when(tail_tile)
    def _():
        pltpu.sync_copy(x_hbm.at[pl.ds(9600, 400)],
                        agg_sh.at[pl.ds(9600, 400)])
        pltpu.sync_copy(rows_v.at[0], agg_sh.at[pl.ds(10000, K)])
        pltpu.sync_copy(rows_v.at[0, pl.ds(0, 112)],
                        agg_sh.at[pl.ds(10128, 112)])

    @pl.when(~is_x_tile)
    def _():
        for q in range(zrows // K):
            pltpu.sync_copy(rows_v.at[0],
                            agg_sh.at[pl.ds(sid * zrows + q * K, K)])

    plsc.subcore_barrier()

    def gather(slot, t, rb, sem):
        return pltpu.async_copy(
            x_hbm.at[seg_v.at[slot, 0, pl.ds(t * K, K)]],
            rows_v.at[rb], sem)

    def scatter(slot, t, rb):
        pltpu.sync_copy(rows_v.at[rb],
                        agg_sh.at[seg_v.at[slot, 1, pl.ds(t * K, K)]],
                        add=True)

    seg_dma(0, 0, isem0).wait()
    seg_dma(1, 1, isem1).start()

    for u in range(NSEG):
        b = u % 2
        if u > 0:
            seg_dma(u, b, isems[b]).wait()
            if u + 1 < NSEG:
                seg_dma(u + 1, 1 - b, isems[1 - b]).start()
        gather(b, 0, 0, gsem0)
        gather(b, 1, 1, gsem1)

        @pl.loop(0, SEG_CH[u] // 2 - 1)
        def _(it):
            for rb in range(2):
                t = 2 * it + rb
                pltpu.make_async_copy(
                    x_hbm.at[seg_v.at[b, 0, pl.ds(t * K, K)]],
                    rows_v.at[rb], gsems[rb]).wait()
                scatter(b, t, rb)
                gather(b, t + 2, rb, gsems[rb])

        for rb in range(2):
            t = SEG_CH[u] - 2 + rb
            pltpu.make_async_copy(
                x_hbm.at[seg_v.at[b, 0, pl.ds(t * K, K)]],
                rows_v.at[rb], gsems[rb]).wait()
            scatter(b, t, rb)

    q = sid * NC + cid

    @pl.when(sid < (XTRA // K) // NC)
    def _():
        pltpu.make_async_copy(
            ei_hbm.at[:, pl.ds(NW * RPW + q * K, K)],
            seg_v.at[0, :, pl.ds(0, K)], isem0).start()
        pltpu.make_async_copy(
            ei_hbm.at[:, pl.ds(NW * RPW + q * K, K)],
            seg_v.at[0, :, pl.ds(0, K)], isem0).wait()
        pltpu.make_async_copy(
            x_hbm.at[seg_v.at[0, 0, pl.ds(0, K)]],
            rows_v.at[0], gsem0).start()
        pltpu.make_async_copy(
            x_hbm.at[seg_v.at[0, 0, pl.ds(0, K)]],
            rows_v.at[0], gsem0).wait()
        scatter(0, 0, 0)

    plsc.subcore_barrier()

    pltpu.sync_copy(agg_sh.at[pl.ds(sid * zrows, zrows)],
                    out_hbm.at[cid, pl.ds(sid * zrows, zrows)])


def _tc_body(agg_ref, batch_ref, W1_ref, b1_ref, W2_ref, b2_ref,
             Wlin_ref, blin_ref, out_ref, pool_acc):
    i = pl.program_id(0)
    h = agg_ref[0] + agg_ref[1]
    z = jnp.dot(h, W1_ref[...], preferred_element_type=jnp.float32)
    z = jnp.maximum(z + b1_ref[...], 0.0)
    h2 = jnp.dot(z, W2_ref[...],
                 preferred_element_type=jnp.float32) + b2_ref[...]
    b = batch_ref[0]
    gids = lax.broadcasted_iota(jnp.int32, (G, BLK), 0)
    mask = (gids == b).astype(jnp.float32)
    seg = jnp.dot(mask, h2, preferred_element_type=jnp.float32,
                  precision=lax.Precision.HIGHEST)

    @pl.when(i == 0)
    def _():
        pool_acc[...] = jnp.zeros_like(pool_acc)
        out_ref[...] = jnp.zeros_like(out_ref)

    pool_acc[...] += seg

    @pl.when(i == pl.num_programs(0) - 1)
    def _():
        out_ref[...] = jnp.dot(
            pool_acc[...], Wlin_ref[...],
            preferred_element_type=jnp.float32) + blin_ref[...]


_tc_call = pl.pallas_call(
    _tc_body,
    grid=(N_BLOCKS,),
    in_specs=[
        pl.BlockSpec((NC, BLK, D), lambda i: (0, i, 0)),
        pl.BlockSpec((1, 1, BLK), lambda i: (i, 0, 0)),
        pl.BlockSpec((D, D), lambda i: (0, 0)),
        pl.BlockSpec((1, D), lambda i: (0, 0)),
        pl.BlockSpec((D, D), lambda i: (0, 0)),
        pl.BlockSpec((1, D), lambda i: (0, 0)),
        pl.BlockSpec((D, 1), lambda i: (0, 0)),
        pl.BlockSpec((1, 1), lambda i: (0, 0)),
    ],
    out_specs=pl.BlockSpec((G, 1), lambda i: (0, 0)),
    out_shape=jax.ShapeDtypeStruct((G, 1), jnp.float32),
    scratch_shapes=[pltpu.VMEM((G, D), jnp.float32)],
)


def kernel(x, edge_index, batch, W1, b1, W2, b2, Wlin, blin):
    ei = edge_index.astype(jnp.int32)
    agg = _sc_scatter_add(x, ei)

    batch_p = jnp.pad(batch.astype(jnp.int32), (0, NPR - N_NODES),
                      constant_values=G)
    batch3 = batch_p.reshape(N_BLOCKS, 1, BLK)
    out = _tc_call(agg, batch3, W1, b1.reshape(1, D), W2,
                   b2.reshape(1, D), Wlin, blin.reshape(1, 1))
    return out


# speedup vs baseline: 1.0480x; 1.0356x over previous
import functools
import jax
import jax.numpy as jnp
from jax import lax
from jax.experimental import pallas as pl
from jax.experimental.pallas import tpu as pltpu
from jax.experimental.pallas import tpu_sc as plsc

N_NODES = 10000
N_EDGES = 320000
D = 128
G = 128

NC = 2
NS = 16
NW = NC * NS

K = 128
NCH = 80
E_PER_W = NCH * K
E_PAD = NW * E_PER_W
NPR = 10240

NSEG = 4
SEGW = 2560
RPW = 78 * K
SEG_CH = (20, 20, 20, 18)
SEG_LEN = tuple(c * K for c in SEG_CH)
XTRA = N_EDGES - NW * RPW

BLK = 2048
N_BLOCKS = NPR // BLK


_sc_mesh = plsc.VectorSubcoreMesh(core_axis_name="c", subcore_axis_name="s")


@functools.partial(
    pl.kernel,
    out_type=jax.ShapeDtypeStruct((NC, NPR, D), jnp.float32),
    mesh=_sc_mesh,
    scratch_types=[
        pltpu.VMEM((2, 2, SEGW), jnp.int32),
        pltpu.VMEM((2, K, D), jnp.float32),
        pltpu.VMEM_SHARED((NPR, D), jnp.float32),
        pltpu.SemaphoreType.DMA,
        pltpu.SemaphoreType.DMA,
        pltpu.SemaphoreType.DMA,
        pltpu.SemaphoreType.DMA,
    ],
)
def _sc_scatter_add(x_hbm, ei_hbm, out_hbm,
                    seg_v, rows_v, agg_sh, gsem0, gsem1, isem0, isem1):
    cid = lax.axis_index("c")
    sid = lax.axis_index("s")
    wid = cid * NS + sid
    gsems = (gsem0, gsem1)
    isems = (isem0, isem1)
    base = wid * RPW

    def seg_dma(u, slot, sem):
        n = SEG_LEN[u]
        return pltpu.make_async_copy(
            ei_hbm.at[:, pl.ds(base + u * SEGW, n)],
            seg_v.at[slot, :, pl.ds(0, n)], sem)

    seg_dma(0, 0, isem0).start()

    @pl.loop(0, K)
    def _(r):
        for c in range(D // 16):
            rows_v[0, r, pl.ds(c * 16, 16)] = jnp.zeros((16,), jnp.float32)

    zrows = NPR // NS
    half = NS // 2
    xlo = cid * half
    is_x_tile = (sid >= xlo) & (sid < xlo + half)
    tail_tile = (cid == 1) & (sid == NS - 1)

    @pl.when(is_x_tile & ~tail_tile)
    def _():
        pltpu.sync_copy(x_hbm.at[pl.ds(sid * zrows, zrows)],
                        agg_sh.at[pl.ds(sid * zrows, zrows)])

    @pl.when(tail_tile)
    def _():
        pltpu.sync_copy(x_hbm.at[pl.ds(9600, 400)],
                        agg_sh.at[pl.ds(9600, 400)])
        pltpu.sync_copy(rows_v.at[0], agg_sh.at[pl.ds(10000, K)])
        pltpu.sync_copy(rows_v.at[0, pl.ds(0, 112)],
                        agg_sh.at[pl.ds(10128, 112)])

    @pl.when(~is_x_tile)
    def _():
        for q in range(zrows // K):
            pltpu.sync_copy(rows_v.at[0],
                            agg_sh.at[pl.ds(sid * zrows + q * K, K)])

    plsc.subcore_barrier()

    def gather(slot, t, rb, sem):
        return pltpu.async_copy(
            x_hbm.at[seg_v.at[slot, 0, pl.ds(t * K, K)]],
            rows_v.at[rb], sem)

    def scatter(slot, t, rb):
        pltpu.sync_copy(rows_v.at[rb],
                        agg_sh.at[seg_v.at[slot, 1, pl.ds(t * K, K)]],
                        add=True)

    seg_dma(0, 0, isem0).wait()
    seg_dma(1, 1, isem1).start()
    gather(0, 0, 0, gsem0)
    gather(0, 1, 1, gsem1)

    for u in range(NSEG):
        b = u % 2
        if 0 < u < NSEG - 1:
            seg_dma(u + 1, 1 - b, isems[1 - b]).start()
        nch = SEG_CH[u]

        @pl.loop(0, (nch - 2) // 2)
        def _(it):
            for rb in range(2):
                t = 2 * it + rb
                pltpu.make_async_copy(
                    x_hbm.at[seg_v.at[b, 0, pl.ds(t * K, K)]],
                    rows_v.at[rb], gsems[rb]).wait()
                scatter(b, t, rb)
                gather(b, t + 2, rb, gsems[rb])

        if u + 1 < NSEG:
            seg_dma(u + 1, 1 - b, isems[1 - b]).wait()
            for rb in range(2):
                t = nch - 2 + rb
                pltpu.make_async_copy(
                    x_hbm.at[seg_v.at[b, 0, pl.ds(t * K, K)]],
                    rows_v.at[rb], gsems[rb]).wait()
                scatter(b, t, rb)
                gather(1 - b, rb, rb, gsems[rb])
        else:
            for rb in range(2):
                t = nch - 2 + rb
                pltpu.make_async_copy(
                    x_hbm.at[seg_v.at[b, 0, pl.ds(t * K, K)]],
                    rows_v.at[rb], gsems[rb]).wait()
                scatter(b, t, rb)

    q = sid * NC + cid

    @pl.when(sid < (XTRA // K) // NC)
    def _():
        pltpu.make_async_copy(
            ei_hbm.at[:, pl.ds(NW * RPW + q * K, K)],
            seg_v.at[0, :, pl.ds(0, K)], isem0).start()
        pltpu.make_async_copy(
            ei_hbm.at[:, pl.ds(NW * RPW + q * K, K)],
            seg_v.at[0, :, pl.ds(0, K)], isem0).wait()
        pltpu.make_async_copy(
            x_hbm.at[seg_v.at[0, 0, pl.ds(0, K)]],
            rows_v.at[0], gsem0).start()
        pltpu.make_async_copy(
            x_hbm.at[seg_v.at[0, 0, pl.ds(0, K)]],
            rows_v.at[0], gsem0).wait()
        scatter(0, 0, 0)

    plsc.subcore_barrier()

    pltpu.sync_copy(agg_sh.at[pl.ds(sid * zrows, zrows)],
                    out_hbm.at[cid, pl.ds(sid * zrows, zrows)])


def _tc_body(agg_ref, batch_ref, W1_ref, b1_ref, W2_ref, b2_ref,
             Wlin_ref, blin_ref, out_ref, pool_acc):
    i = pl.program_id(0)
    h = agg_ref[0] + agg_ref[1]
    z = jnp.dot(h, W1_ref[...], preferred_element_type=jnp.float32)
    z = jnp.maximum(z + b1_ref[...], 0.0)
    h2 = jnp.dot(z, W2_ref[...],
                 preferred_element_type=jnp.float32) + b2_ref[...]
    b = batch_ref[0]
    gids = lax.broadcasted_iota(jnp.int32, (G, BLK), 0)
    mask = (gids == b).astype(jnp.float32)
    seg = jnp.dot(mask, h2, preferred_element_type=jnp.float32,
                  precision=lax.Precision.HIGHEST)

    @pl.when(i == 0)
    def _():
        pool_acc[...] = jnp.zeros_like(pool_acc)
        out_ref[...] = jnp.zeros_like(out_ref)

    pool_acc[...] += seg

    @pl.when(i == pl.num_programs(0) - 1)
    def _():
        out_ref[...] = jnp.dot(
            pool_acc[...], Wlin_ref[...],
            preferred_element_type=jnp.float32) + blin_ref[...]


_tc_call = pl.pallas_call(
    _tc_body,
    grid=(N_BLOCKS,),
    in_specs=[
        pl.BlockSpec((NC, BLK, D), lambda i: (0, i, 0)),
        pl.BlockSpec((1, 1, BLK), lambda i: (i, 0, 0)),
        pl.BlockSpec((D, D), lambda i: (0, 0)),
        pl.BlockSpec((1, D), lambda i: (0, 0)),
        pl.BlockSpec((D, D), lambda i: (0, 0)),
        pl.BlockSpec((1, D), lambda i: (0, 0)),
        pl.BlockSpec((D, 1), lambda i: (0, 0)),
        pl.BlockSpec((1, 1), lambda i: (0, 0)),
    ],
    out_specs=pl.BlockSpec((G, 1), lambda i: (0, 0)),
    out_shape=jax.ShapeDtypeStruct((G, 1), jnp.float32),
    scratch_shapes=[pltpu.VMEM((G, D), jnp.float32)],
)


def kernel(x, edge_index, batch, W1, b1, W2, b2, Wlin, blin):
    ei = edge_index.astype(jnp.int32)
    agg = _sc_scatter_add(x, ei)

    batch_p = jnp.pad(batch.astype(jnp.int32), (0, NPR - N_NODES),
                      constant_values=G)
    batch3 = batch_p.reshape(N_BLOCKS, 1, BLK)
    out = _tc_call(agg, batch3, W1, b1.reshape(1, D), W2,
                   b2.reshape(1, D), Wlin, blin.reshape(1, 1))
    return out


# speedup vs baseline: 1.0525x; 1.0043x over previous
import functools
import jax
import jax.numpy as jnp
from jax import lax
from jax.experimental import pallas as pl
from jax.experimental.pallas import tpu as pltpu
from jax.experimental.pallas import tpu_sc as plsc

N_NODES = 10000
N_EDGES = 320000
D = 128
G = 128

NC = 2
NS = 16
NW = NC * NS

K = 128
NCH = 80
E_PER_W = NCH * K
E_PAD = NW * E_PER_W
NPR = 10240

NSEG = 4
SEGW = 2560
RPW = 78 * K
SEG_CH = (20, 20, 20, 18)
SEG_LEN = tuple(c * K for c in SEG_CH)
XTRA = N_EDGES - NW * RPW

BLK = 2048
N_BLOCKS = NPR // BLK


_sc_mesh = plsc.VectorSubcoreMesh(core_axis_name="c", subcore_axis_name="s")


@functools.partial(
    pl.kernel,
    out_type=jax.ShapeDtypeStruct((NC, NPR, D), jnp.float32),
    mesh=_sc_mesh,
    scratch_types=[
        pltpu.VMEM((2, 2, SEGW), jnp.int32),
        pltpu.VMEM((2, K, D), jnp.float32),
        pltpu.VMEM_SHARED((NPR, D), jnp.float32),
        pltpu.SemaphoreType.DMA,
        pltpu.SemaphoreType.DMA,
        pltpu.SemaphoreType.DMA,
        pltpu.SemaphoreType.DMA,
    ],
)
def _sc_scatter_add(x_hbm, ei_hbm, out_hbm,
                    seg_v, rows_v, agg_sh, gsem0, gsem1, isem0, isem1):
    cid = lax.axis_index("c")
    sid = lax.axis_index("s")
    wid = cid * NS + sid
    gsems = (gsem0, gsem1)
    isems = (isem0, isem1)
    base = wid * RPW

    def seg_dma(u, slot, sem):
        n = SEG_LEN[u]
        return pltpu.make_async_copy(
            ei_hbm.at[:, pl.ds(base + u * SEGW, n)],
            seg_v.at[slot, :, pl.ds(0, n)], sem)

    seg_dma(0, 0, isem0).start()

    @pl.loop(0, K)
    def _(r):
        for c in range(D // 16):
            rows_v[0, r, pl.ds(c * 16, 16)] = jnp.zeros((16,), jnp.float32)

    zrows = NPR // NS
    half = NS // 2
    xlo = cid * half
    is_x_tile = (sid >= xlo) & (sid < xlo + half)
    tail_tile = (cid == 1) & (sid == NS - 1)

    @pl.when(is_x_tile & ~tail_tile)
    def _():
        pltpu.sync_copy(x_hbm.at[pl.ds(sid * zrows, zrows)],
                        agg_sh.at[pl.ds(sid * zrows, zrows)])

    @pl.when(tail_tile)
    def _():
        pltpu.sync_copy(x_hbm.at[pl.ds(9600, 400)],
                        agg_sh.at[pl.ds(9600, 400)])
        pltpu.sync_copy(rows_v.at[0], agg_sh.at[pl.ds(10000, K)])
        pltpu.sync_copy(rows_v.at[0, pl.ds(0, 112)],
                        agg_sh.at[pl.ds(10128, 112)])

    @pl.when(~is_x_tile)
    def _():
        for q in range(zrows // K):
            pltpu.sync_copy(rows_v.at[0],
                            agg_sh.at[pl.ds(sid * zrows + q * K, K)])

    plsc.subcore_barrier()

    HK = K // 2

    def gather(slot, t, rb, sem):
        for h2 in range(2):
            pltpu.async_copy(
                x_hbm.at[seg_v.at[slot, 0, pl.ds(t * K + h2 * HK, HK)]],
                rows_v.at[rb, pl.ds(h2 * HK, HK)], sem)

    def gwait(slot, t, rb, sem):
        for h2 in range(2):
            pltpu.make_async_copy(
                x_hbm.at[seg_v.at[slot, 0, pl.ds(t * K + h2 * HK, HK)]],
                rows_v.at[rb, pl.ds(h2 * HK, HK)], sem).wait()

    def scatter(slot, t, rb):
        pltpu.sync_copy(rows_v.at[rb],
                        agg_sh.at[seg_v.at[slot, 1, pl.ds(t * K, K)]],
                        add=True)

    seg_dma(0, 0, isem0).wait()
    seg_dma(1, 1, isem1).start()
    gather(0, 0, 0, gsem0)
    gather(0, 1, 1, gsem1)

    for u in range(NSEG):
        b = u % 2
        if 0 < u < NSEG - 1:
            seg_dma(u + 1, 1 - b, isems[1 - b]).start()
        nch = SEG_CH[u]

        @pl.loop(0, (nch - 2) // 2)
        def _(it):
            for rb in range(2):
                t = 2 * it + rb
                gwait(b, t, rb, gsems[rb])
                scatter(b, t, rb)
                gather(b, t + 2, rb, gsems[rb])

        if u + 1 < NSEG:
            seg_dma(u + 1, 1 - b, isems[1 - b]).wait()
            for rb in range(2):
                t = nch - 2 + rb
                gwait(b, t, rb, gsems[rb])
                scatter(b, t, rb)
                gather(1 - b, rb, rb, gsems[rb])
        else:
            for rb in range(2):
                t = nch - 2 + rb
                gwait(b, t, rb, gsems[rb])
                scatter(b, t, rb)

    q = sid * NC + cid

    @pl.when(sid < (XTRA // K) // NC)
    def _():
        pltpu.make_async_copy(
            ei_hbm.at[:, pl.ds(NW * RPW + q * K, K)],
            seg_v.at[0, :, pl.ds(0, K)], isem0).start()
        pltpu.make_async_copy(
            ei_hbm.at[:, pl.ds(NW * RPW + q * K, K)],
            seg_v.at[0, :, pl.ds(0, K)], isem0).wait()
        pltpu.make_async_copy(
            x_hbm.at[seg_v.at[0, 0, pl.ds(0, K)]],
            rows_v.at[0], gsem0).start()
        pltpu.make_async_copy(
            x_hbm.at[seg_v.at[0, 0, pl.ds(0, K)]],
            rows_v.at[0], gsem0).wait()
        scatter(0, 0, 0)

    plsc.subcore_barrier()

    pltpu.sync_copy(agg_sh.at[pl.ds(sid * zrows, zrows)],
                    out_hbm.at[cid, pl.ds(sid * zrows, zrows)])


def _tc_body(agg_ref, batch_ref, W1_ref, b1_ref, W2_ref, b2_ref,
             Wlin_ref, blin_ref, out_ref, pool_acc):
    i = pl.program_id(0)
    h = agg_ref[0] + agg_ref[1]
    z = jnp.dot(h, W1_ref[...], preferred_element_type=jnp.float32)
    z = jnp.maximum(z + b1_ref[...], 0.0)
    h2 = jnp.dot(z, W2_ref[...],
                 preferred_element_type=jnp.float32) + b2_ref[...]
    b = batch_ref[0]
    gids = lax.broadcasted_iota(jnp.int32, (G, BLK), 0)
    mask = (gids == b).astype(jnp.float32)
    seg = jnp.dot(mask, h2, preferred_element_type=jnp.float32,
                  precision=lax.Precision.HIGHEST)

    @pl.when(i == 0)
    def _():
        pool_acc[...] = jnp.zeros_like(pool_acc)
        out_ref[...] = jnp.zeros_like(out_ref)

    pool_acc[...] += seg

    @pl.when(i == pl.num_programs(0) - 1)
    def _():
        out_ref[...] = jnp.dot(
            pool_acc[...], Wlin_ref[...],
            preferred_element_type=jnp.float32) + blin_ref[...]


_tc_call = pl.pallas_call(
    _tc_body,
    grid=(N_BLOCKS,),
    in_specs=[
        pl.BlockSpec((NC, BLK, D), lambda i: (0, i, 0)),
        pl.BlockSpec((1, 1, BLK), lambda i: (i, 0, 0)),
        pl.BlockSpec((D, D), lambda i: (0, 0)),
        pl.BlockSpec((1, D), lambda i: (0, 0)),
        pl.BlockSpec((D, D), lambda i: (0, 0)),
        pl.BlockSpec((1, D), lambda i: (0, 0)),
        pl.BlockSpec((D, 1), lambda i: (0, 0)),
        pl.BlockSpec((1, 1), lambda i: (0, 0)),
    ],
    out_specs=pl.BlockSpec((G, 1), lambda i: (0, 0)),
    out_shape=jax.ShapeDtypeStruct((G, 1), jnp.float32),
    scratch_shapes=[pltpu.VMEM((G, D), jnp.float32)],
)


def kernel(x, edge_index, batch, W1, b1, W2, b2, Wlin, blin):
    ei = edge_index.astype(jnp.int32)
    agg = _sc_scatter_add(x, ei)

    batch_p = jnp.pad(batch.astype(jnp.int32), (0, NPR - N_NODES),
                      constant_values=G)
    batch3 = batch_p.reshape(N_BLOCKS, 1, BLK)
    out = _tc_call(agg, batch3, W1, b1.reshape(1, D), W2,
                   b2.reshape(1, D), Wlin, blin.reshape(1, 1))
    return out


# speedup vs baseline: 1.0597x; 1.0068x over previous
import functools
import jax
import jax.numpy as jnp
from jax import lax
from jax.experimental import pallas as pl
from jax.experimental.pallas import tpu as pltpu
from jax.experimental.pallas import tpu_sc as plsc

N_NODES = 10000
N_EDGES = 320000
D = 128
G = 128

NC = 2
NS = 16
NW = NC * NS

K = 128
NCH = 80
E_PER_W = NCH * K
E_PAD = NW * E_PER_W
NPR = 10240

NSEG = 3
SEGW = 3328
RPW = 78 * K
SEG_CH = (26, 26, 26)
SEG_LEN = tuple(c * K for c in SEG_CH)
XTRA = N_EDGES - NW * RPW

BLK = 2048
N_BLOCKS = NPR // BLK


_sc_mesh = plsc.VectorSubcoreMesh(core_axis_name="c", subcore_axis_name="s")


@functools.partial(
    pl.kernel,
    out_type=jax.ShapeDtypeStruct((NC, NPR, D), jnp.float32),
    mesh=_sc_mesh,
    scratch_types=[
        pltpu.VMEM((2, 2, SEGW), jnp.int32),
        pltpu.VMEM((2, K, D), jnp.float32),
        pltpu.VMEM_SHARED((NPR, D), jnp.float32),
        pltpu.SemaphoreType.DMA,
        pltpu.SemaphoreType.DMA,
        pltpu.SemaphoreType.DMA,
        pltpu.SemaphoreType.DMA,
    ],
)
def _sc_scatter_add(x_hbm, ei_hbm, out_hbm,
                    seg_v, rows_v, agg_sh, gsem0, gsem1, isem0, isem1):
    cid = lax.axis_index("c")
    sid = lax.axis_index("s")
    wid = cid * NS + sid
    gsems = (gsem0, gsem1)
    isems = (isem0, isem1)
    base = wid * RPW

    def seg_dma(u, slot, sem):
        n = SEG_LEN[u]
        return pltpu.make_async_copy(
            ei_hbm.at[:, pl.ds(base + u * SEGW, n)],
            seg_v.at[slot, :, pl.ds(0, n)], sem)

    seg_dma(0, 0, isem0).start()

    @pl.loop(0, K)
    def _(r):
        for c in range(D // 16):
            rows_v[0, r, pl.ds(c * 16, 16)] = jnp.zeros((16,), jnp.float32)

    zrows = NPR // NS
    half = NS // 2
    xlo = cid * half
    is_x_tile = (sid >= xlo) & (sid < xlo + half)
    tail_tile = (cid == 1) & (sid == NS - 1)

    @pl.when(is_x_tile & ~tail_tile)
    def _():
        pltpu.sync_copy(x_hbm.at[pl.ds(sid * zrows, zrows)],
                        agg_sh.at[pl.ds(sid * zrows, zrows)])

    @pl.when(tail_tile)
    def _():
        pltpu.sync_copy(x_hbm.at[pl.ds(9600, 400)],
                        agg_sh.at[pl.ds(9600, 400)])
        pltpu.sync_copy(rows_v.at[0], agg_sh.at[pl.ds(10000, K)])
        pltpu.sync_copy(rows_v.at[0, pl.ds(0, 112)],
                        agg_sh.at[pl.ds(10128, 112)])

    @pl.when(~is_x_tile)
    def _():
        for q in range(zrows // K):
            pltpu.sync_copy(rows_v.at[0],
                            agg_sh.at[pl.ds(sid * zrows + q * K, K)])

    plsc.subcore_barrier()

    HK = K // 2

    def gather(slot, t, rb, sem):
        for h2 in range(2):
            pltpu.async_copy(
                x_hbm.at[seg_v.at[slot, 0, pl.ds(t * K + h2 * HK, HK)]],
                rows_v.at[rb, pl.ds(h2 * HK, HK)], sem)

    def gwait(slot, t, rb, sem):
        for h2 in range(2):
            pltpu.make_async_copy(
                x_hbm.at[seg_v.at[slot, 0, pl.ds(t * K + h2 * HK, HK)]],
                rows_v.at[rb, pl.ds(h2 * HK, HK)], sem).wait()

    def scatter(slot, t, rb):
        pltpu.sync_copy(rows_v.at[rb],
                        agg_sh.at[seg_v.at[slot, 1, pl.ds(t * K, K)]],
                        add=True)

    seg_dma(0, 0, isem0).wait()
    seg_dma(1, 1, isem1).start()
    gather(0, 0, 0, gsem0)
    gather(0, 1, 1, gsem1)

    for u in range(NSEG):
        b = u % 2
        if 0 < u < NSEG - 1:
            seg_dma(u + 1, 1 - b, isems[1 - b]).start()
        nch = SEG_CH[u]

        @pl.loop(0, (nch - 2) // 2)
        def _(it):
            for rb in range(2):
                t = 2 * it + rb
                gwait(b, t, rb, gsems[rb])
                scatter(b, t, rb)
                gather(b, t + 2, rb, gsems[rb])

        if u + 1 < NSEG:
            seg_dma(u + 1, 1 - b, isems[1 - b]).wait()
            for rb in range(2):
                t = nch - 2 + rb
                gwait(b, t, rb, gsems[rb])
                scatter(b, t, rb)
                gather(1 - b, rb, rb, gsems[rb])
        else:
            for rb in range(2):
                t = nch - 2 + rb
                gwait(b, t, rb, gsems[rb])
                scatter(b, t, rb)

    q = sid * NC + cid

    @pl.when(sid < (XTRA // K) // NC)
    def _():
        pltpu.make_async_copy(
            ei_hbm.at[:, pl.ds(NW * RPW + q * K, K)],
            seg_v.at[0, :, pl.ds(0, K)], isem0).start()
        pltpu.make_async_copy(
            ei_hbm.at[:, pl.ds(NW * RPW + q * K, K)],
            seg_v.at[0, :, pl.ds(0, K)], isem0).wait()
        pltpu.make_async_copy(
            x_hbm.at[seg_v.at[0, 0, pl.ds(0, K)]],
            rows_v.at[0], gsem0).start()
        pltpu.make_async_copy(
            x_hbm.at[seg_v.at[0, 0, pl.ds(0, K)]],
            rows_v.at[0], gsem0).wait()
        scatter(0, 0, 0)

    plsc.subcore_barrier()

    pltpu.sync_copy(agg_sh.at[pl.ds(sid * zrows, zrows)],
                    out_hbm.at[cid, pl.ds(sid * zrows, zrows)])


def _tc_body(agg_ref, batch_ref, W1_ref, b1_ref, W2_ref, b2_ref,
             Wlin_ref, blin_ref, out_ref, pool_acc):
    i = pl.program_id(0)
    h = agg_ref[0] + agg_ref[1]
    z = jnp.dot(h, W1_ref[...], preferred_element_type=jnp.float32)
    z = jnp.maximum(z + b1_ref[...], 0.0)
    h2 = jnp.dot(z, W2_ref[...],
                 preferred_element_type=jnp.float32) + b2_ref[...]
    b = batch_ref[0]
    gids = lax.broadcasted_iota(jnp.int32, (G, BLK), 0)
    mask = (gids == b).astype(jnp.float32)
    seg = jnp.dot(mask, h2, preferred_element_type=jnp.float32,
                  precision=lax.Precision.HIGHEST)

    @pl.when(i == 0)
    def _():
        pool_acc[...] = jnp.zeros_like(pool_acc)
        out_ref[...] = jnp.zeros_like(out_ref)

    pool_acc[...] += seg

    @pl.when(i == pl.num_programs(0) - 1)
    def _():
        out_ref[...] = jnp.dot(
            pool_acc[...], Wlin_ref[...],
            preferred_element_type=jnp.float32) + blin_ref[...]


_tc_call = pl.pallas_call(
    _tc_body,
    grid=(N_BLOCKS,),
    in_specs=[
        pl.BlockSpec((NC, BLK, D), lambda i: (0, i, 0)),
        pl.BlockSpec((1, 1, BLK), lambda i: (i, 0, 0)),
        pl.BlockSpec((D, D), lambda i: (0, 0)),
        pl.BlockSpec((1, D), lambda i: (0, 0)),
        pl.BlockSpec((D, D), lambda i: (0, 0)),
        pl.BlockSpec((1, D), lambda i: (0, 0)),
        pl.BlockSpec((D, 1), lambda i: (0, 0)),
        pl.BlockSpec((1, 1), lambda i: (0, 0)),
    ],
    out_specs=pl.BlockSpec((G, 1), lambda i: (0, 0)),
    out_shape=jax.ShapeDtypeStruct((G, 1), jnp.float32),
    scratch_shapes=[pltpu.VMEM((G, D), jnp.float32)],
)


def kernel(x, edge_index, batch, W1, b1, W2, b2, Wlin, blin):
    ei = edge_index.astype(jnp.int32)
    agg = _sc_scatter_add(x, ei)

    batch_p = jnp.pad(batch.astype(jnp.int32), (0, NPR - N_NODES),
                      constant_values=G)
    batch3 = batch_p.reshape(N_BLOCKS, 1, BLK)
    out = _tc_call(agg, batch3, W1, b1.reshape(1, D), W2,
                   b2.reshape(1, D), Wlin, blin.reshape(1, 1))
    return out
